# Initial kernel scaffold; baseline (speedup 1.0000x reference)
import jax
import jax.numpy as jnp
from jax.experimental import pallas as pl


def kernel(X, weight):
    raise NotImplementedError("write your pallas kernel here")


# speedup vs baseline: 2.2089x; 2.2089x over previous
import functools

import jax
import jax.numpy as jnp
from jax import lax
from jax.experimental import pallas as pl
from jax.experimental.pallas import tpu as pltpu
from jax.experimental.pallas import tpu_sc as plsc

VOCAB = 100000
DIM = 128
DROPOUT = 0.1
NC = 2
NS = 16
NW = NC * NS
LANES = 16
CHUNK = 128


def _drop_lookup(x_tiles, weight, scale16):
    _, n_chunks, _ = x_tiles.shape
    b_per_w = n_chunks * CHUNK
    total = NW * b_per_w
    mesh = plsc.VectorSubcoreMesh(core_axis_name="c", subcore_axis_name="s")

    @functools.partial(
        pl.kernel,
        mesh=mesh,
        compiler_params=pltpu.CompilerParams(use_tc_tiling_on_sc=False),
        out_type=jax.ShapeDtypeStruct((total, DIM), jnp.float32),
        scratch_types=[
            pltpu.VMEM((n_chunks, CHUNK), jnp.int32),
            pltpu.VMEM((CHUNK, DIM), jnp.float32),
            pltpu.VMEM((CHUNK, LANES), jnp.float32),
            pltpu.SemaphoreType.DMA,
            pltpu.SemaphoreType.DMA,
        ],
    )
    def body(x_hbm, w_hbm, s_hbm, out_hbm, idx_v, rows_v, sv_v, sem_w, sem_s):
        wid = lax.axis_index("s") * NC + lax.axis_index("c")
        base = wid * b_per_w
        pltpu.sync_copy(x_hbm.at[wid], idx_v)

        def chunk_body(j, carry):
            gw = pltpu.async_copy(w_hbm.at[idx_v.at[j]], rows_v, sem_w)
            gs = pltpu.async_copy(s_hbm.at[idx_v.at[j]], sv_v, sem_s)
            gw.wait()
            gs.wait()

            def row_body(i, c):
                s = sv_v[i]
                for k in range(DIM // LANES):
                    sl = pl.ds(k * LANES, LANES)
                    rows_v[i, sl] = rows_v[i, sl] * s
                return c

            lax.fori_loop(0, CHUNK, row_body, 0, unroll=2)
            pltpu.sync_copy(rows_v, out_hbm.at[pl.ds(base + j * CHUNK, CHUNK)])
            return carry

        lax.fori_loop(0, n_chunks, chunk_body, 0)

    return body(x_tiles, weight, scale16)


def kernel(X, weight):
    batch, hist = X.shape
    total = batch * hist
    n_chunks = total // (NW * CHUNK)
    mask_key = jax.random.fold_in(jax.random.key(0), 1)
    keep = jax.random.bernoulli(mask_key, 1.0 - DROPOUT, (VOCAB, 1)).astype(
        weight.dtype
    )
    scale16 = jnp.broadcast_to(keep / (1.0 - DROPOUT), (VOCAB, LANES))
    x_tiles = X.reshape(NW, n_chunks, CHUNK)
    out = _drop_lookup(x_tiles, weight, scale16)
    return out.reshape(batch, hist, DIM)


# speedup vs baseline: 10.3704x; 4.6949x over previous
import functools

import jax
import jax.numpy as jnp
import numpy as np
from jax import lax
from jax.experimental import pallas as pl
from jax.experimental.pallas import tpu as pltpu
from jax.experimental.pallas import tpu_sc as plsc

VOCAB = 100000
DIM = 128
DROPOUT = 0.1
NC = 2
NS = 16
NW = NC * NS
LANES = 16
CHUNK = 64
NBUF = 6
NBITS = VOCAB // 32
SCALE = float(np.float32(1.0) / np.float32(1.0 - DROPOUT))


def _drop_lookup(x_tiles, weight, bits):
    _, n_chunks, _ = x_tiles.shape
    b_per_w = n_chunks * CHUNK
    total = NW * b_per_w
    mesh = plsc.VectorSubcoreMesh(core_axis_name="c", subcore_axis_name="s")

    @functools.partial(
        pl.kernel,
        mesh=mesh,
        compiler_params=pltpu.CompilerParams(needs_layout_passes=False),
        out_type=jax.ShapeDtypeStruct((total, DIM), jnp.float32),
        scratch_types=(
            [
                pltpu.VMEM((n_chunks, CHUNK), jnp.int32),
                pltpu.VMEM((NBITS,), jnp.int32),
            ]
            + [pltpu.VMEM((CHUNK, DIM), jnp.float32)] * NBUF
            + [pltpu.SemaphoreType.DMA] * (2 * NBUF)
        ),
    )
    def body(x_hbm, w_hbm, b_hbm, out_hbm, idx_v, bits_v, *ring):
        rows_bufs = ring[:NBUF]
        sem_g = ring[NBUF : 2 * NBUF]
        sem_o = ring[2 * NBUF :]
        wid = lax.axis_index("s") * NC + lax.axis_index("c")
        base = wid * b_per_w
        pltpu.sync_copy(x_hbm.at[wid], idx_v)
        bits_copy = pltpu.async_copy(b_hbm, bits_v, sem_o[0])
        zero16 = lax.iota(jnp.int32, LANES) * 0
        bufs = tuple(zip(rows_bufs, sem_g, sem_o))

        def gather_start(j, rows, sem_g):
            pltpu.async_copy(w_hbm.at[idx_v.at[j]], rows, sem_g)

        def out_slice(j):
            return out_hbm.at[pl.ds(base + j * CHUNK, CHUNK)]

        def scale_rows(j, rows):
            def group_body(g, c):
                tvec = idx_v[j, pl.ds(g * LANES, LANES)]
                words = plsc.load_gather(
                    bits_v, [lax.shift_right_logical(tvec, 5)]
                )
                bit = lax.shift_right_logical(words, tvec & 31) & 1
                svec = lax.convert_element_type(bit, jnp.float32) * SCALE
                for r in range(LANES):
                    s = jnp.take(svec, zero16 + r)
                    row = g * LANES + r
                    for k in range(DIM // LANES):
                        sl = pl.ds(k * LANES, LANES)
                        rows[row, sl] = rows[row, sl] * s
                return c

            lax.fori_loop(0, CHUNK // LANES, group_body, 0)

        for b, (rows, sg, _) in enumerate(bufs):
            gather_start(b, rows, sg)
        bits_copy.wait()

        def ring_body(t, carry):
            for b, (rows, sg, so) in enumerate(bufs):
                j = NBUF * t + b

                @pl.when(j < n_chunks)
                def _():
                    pltpu.make_async_copy(
                        w_hbm.at[idx_v.at[j]], rows, sg
                    ).wait()
                    scale_rows(j, rows)
                    pltpu.async_copy(rows, out_slice(j), so)

                    @pl.when(j + NBUF < n_chunks)
                    def _():
                        pltpu.make_async_copy(rows, out_slice(j), so).wait()
                        gather_start(j + NBUF, rows, sg)

            return carry

        lax.fori_loop(0, -(-n_chunks // NBUF), ring_body, 0)
        for b, (rows, _, so) in enumerate(bufs):
            j = n_chunks - NBUF + b
            pltpu.make_async_copy(rows, out_slice(j), so).wait()

    return body(x_tiles, weight, bits)


def kernel(X, weight):
    batch, hist = X.shape
    total = batch * hist
    n_chunks = total // (NW * CHUNK)
    mask_key = jax.random.fold_in(jax.random.key(0), 1)
    keep = jax.random.bernoulli(mask_key, 1.0 - DROPOUT, (VOCAB, 1))
    bits = lax.bitcast_convert_type(
        jnp.sum(
            keep.reshape(NBITS, 32).astype(jnp.uint32)
            << jnp.arange(32, dtype=jnp.uint32),
            axis=1,
            dtype=jnp.uint32,
        ),
        jnp.int32,
    )
    x_tiles = X.T.reshape(NW, n_chunks, CHUNK)
    out = _drop_lookup(x_tiles, weight, bits)
    return out.reshape(hist, batch, DIM).transpose(1, 0, 2)
